# baseline (speedup 1.0000x reference)
import functools

import jax
import jax.numpy as jnp
from jax import lax
from jax.experimental import pallas as pl
from jax.experimental.pallas import tpu as pltpu

N = 10000
E = 320000
H = 128
HH = 64
A = 100
B_E = 8000


def _embed_h_body(x_ref, w_ref, b_ref, o_ref):
    o_ref[...] = jnp.dot(x_ref[...], w_ref[...], preferred_element_type=jnp.float32) + b_ref[...]


def _node_mm_body(x_ref, w_ref, b_ref, ah_ref, bh_ref, dh_ref, eh_ref):
    y = jnp.dot(x_ref[...], w_ref[...], preferred_element_type=jnp.float32) + b_ref[...]
    ah_ref[...] = y[:, 0:H]
    for t_ref, off in ((bh_ref, H), (dh_ref, 2 * H), (eh_ref, 3 * H)):
        t_ref[0:N] = y[:, off:off + HH]
        t_ref[N:2 * N] = y[:, off + HH:off + 2 * HH]


def _node_update_body(ah_ref, num_ref, den_ref, hres_ref, o_ref):
    hn = ah_ref[...] + num_ref[...] / (den_ref[...] + 1e-6)
    mu = jnp.mean(hn, axis=0, keepdims=True)
    var = jnp.mean((hn - mu) ** 2, axis=0, keepdims=True)
    o_ref[...] = hres_ref[...] + jax.nn.relu((hn - mu) * lax.rsqrt(var + 1e-5))


def _edge_embed_body(e_ref, we_ref, be_ref, w2_ref, b2_ref, e0_ref, ce_ref):
    e0 = jnp.dot(e_ref[...], we_ref[...], preferred_element_type=jnp.float32) + be_ref[...]
    e0_ref[...] = e0
    ce = jnp.dot(e0, w2_ref[...], preferred_element_type=jnp.float32) + b2_ref[...]
    ce_ref[0] = ce[:, 0:HH]
    ce_ref[1] = ce[:, HH:H]


def _edge_update_body(eres_ref, epa_ref, epb_ref, mu_ref, rstd_ref, w2_ref, b2_ref,
                      eo_ref, ce_ref):
    ep = jnp.concatenate([epa_ref[0], epb_ref[0]], axis=-1)
    eo = eres_ref[...] + jax.nn.relu((ep - mu_ref[...]) * rstd_ref[...])
    eo_ref[...] = eo
    ce = jnp.dot(eo, w2_ref[...], preferred_element_type=jnp.float32) + b2_ref[...]
    ce_ref[0] = ce[:, 0:HH]
    ce_ref[1] = ce[:, HH:H]


def _bilin_s_body(x_ref, wa_ref, ba_ref, s_ref):
    z = jnp.dot(x_ref[...], wa_ref[...], preferred_element_type=jnp.float32) + ba_ref[...]
    m = jnp.max(z, axis=-1, keepdims=True)
    ez = jnp.exp(z - m)
    s_ref[...] = ez / jnp.sum(ez, axis=-1, keepdims=True)


def _bilin_update_body(h_ref, s_ref, st_ref, o_ref):
    m = jnp.dot(st_ref[...], h_ref[...], preferred_element_type=jnp.float32)
    o_ref[...] = h_ref[...] + jnp.dot(s_ref[...], m, preferred_element_type=jnp.float32)


def _readout_pq_body(h_ref, w0_ref, p_ref, q_ref):
    p = jnp.dot(h_ref[...], w0_ref[0:H], preferred_element_type=jnp.float32)
    q = jnp.dot(h_ref[...], w0_ref[H:2 * H], preferred_element_type=jnp.float32)
    p_ref[0:N] = p[:, 0:HH]
    p_ref[N:2 * N] = p[:, HH:H]
    q_ref[0:N] = q[:, 0:HH]
    q_ref[N:2 * N] = q[:, HH:H]


def _mlp_body(xa_ref, xb_ref, b0_ref, w1_ref, b1_ref, w2_ref, b2_ref, o_ref):
    x = jax.nn.relu(jnp.concatenate([xa_ref[0], xb_ref[0]], axis=-1) + b0_ref[...])
    y = jax.nn.relu(jnp.dot(x, w1_ref[...], preferred_element_type=jnp.float32) + b1_ref[...])
    o_ref[...] = jnp.dot(y, w2_ref[...], preferred_element_type=jnp.float32) + b2_ref[...]


def _full(shape):
    return pl.BlockSpec(shape, lambda: tuple(0 for _ in shape))


def _embed_h(h, W, b):
    return pl.pallas_call(
        _embed_h_body,
        out_shape=jax.ShapeDtypeStruct((N, H), jnp.float32),
    )(h, W, b.reshape(1, H))


def _node_mm(x, Wstk, bstk):
    return pl.pallas_call(
        _node_mm_body,
        out_shape=[
            jax.ShapeDtypeStruct((N, H), jnp.float32),
            jax.ShapeDtypeStruct((2 * N, HH), jnp.float32),
            jax.ShapeDtypeStruct((2 * N, HH), jnp.float32),
            jax.ShapeDtypeStruct((2 * N, HH), jnp.float32),
        ],
    )(x, Wstk, bstk)


def _node_update(Ah, num, den, hres):
    return pl.pallas_call(
        _node_update_body,
        out_shape=jax.ShapeDtypeStruct((N, H), jnp.float32),
    )(Ah, num, den, hres)


def _edge_embed(e, Wemb, bemb, W2, b2):
    g = E // B_E
    return pl.pallas_call(
        _edge_embed_body,
        grid=(g,),
        in_specs=[
            pl.BlockSpec((B_E, 16), lambda i: (i, 0)),
            pl.BlockSpec((16, H), lambda i: (0, 0)),
            pl.BlockSpec((1, H), lambda i: (0, 0)),
            pl.BlockSpec((H, H), lambda i: (0, 0)),
            pl.BlockSpec((1, H), lambda i: (0, 0)),
        ],
        out_specs=[
            pl.BlockSpec((B_E, H), lambda i: (i, 0)),
            pl.BlockSpec((2, B_E, HH), lambda i: (0, i, 0)),
        ],
        out_shape=[
            jax.ShapeDtypeStruct((E, H), jnp.float32),
            jax.ShapeDtypeStruct((2, E, HH), jnp.float32),
        ],
    )(e, Wemb, bemb.reshape(1, H), W2, b2.reshape(1, H))


def _edge_update(eres, ep, mu, rstd, W2n, b2n):
    g = E // B_E
    return pl.pallas_call(
        _edge_update_body,
        grid=(g,),
        in_specs=[
            pl.BlockSpec((B_E, H), lambda i: (i, 0)),
            pl.BlockSpec((1, B_E, HH), lambda i: (0, i, 0)),
            pl.BlockSpec((1, B_E, HH), lambda i: (1, i, 0)),
            pl.BlockSpec((1, H), lambda i: (0, 0)),
            pl.BlockSpec((1, H), lambda i: (0, 0)),
            pl.BlockSpec((H, H), lambda i: (0, 0)),
            pl.BlockSpec((1, H), lambda i: (0, 0)),
        ],
        out_specs=[
            pl.BlockSpec((B_E, H), lambda i: (i, 0)),
            pl.BlockSpec((2, B_E, HH), lambda i: (0, i, 0)),
        ],
        out_shape=[
            jax.ShapeDtypeStruct((E, H), jnp.float32),
            jax.ShapeDtypeStruct((2, E, HH), jnp.float32),
        ],
    )(eres, ep, ep, mu, rstd, W2n, b2n.reshape(1, H))


def _bilinear(h, Wa, ba):
    s = pl.pallas_call(
        _bilin_s_body,
        out_shape=jax.ShapeDtypeStruct((N, A), jnp.float32),
    )(h, Wa, ba.reshape(1, A))
    h_out = pl.pallas_call(
        _bilin_update_body,
        out_shape=jax.ShapeDtypeStruct((N, H), jnp.float32),
    )(h, s, s.T)
    return h_out, s


def _readout_pq(h, W0):
    return pl.pallas_call(
        _readout_pq_body,
        out_shape=[
            jax.ShapeDtypeStruct((2 * N, HH), jnp.float32),
            jax.ShapeDtypeStruct((2 * N, HH), jnp.float32),
        ],
    )(h, W0)


def _readout_mlp(xp, b0, W1, b1, W2, b2):
    g = E // B_E
    return pl.pallas_call(
        _mlp_body,
        grid=(g,),
        in_specs=[
            pl.BlockSpec((1, B_E, HH), lambda i: (0, i, 0)),
            pl.BlockSpec((1, B_E, HH), lambda i: (1, i, 0)),
            pl.BlockSpec((1, H), lambda i: (0, 0)),
            pl.BlockSpec((H, HH), lambda i: (0, 0)),
            pl.BlockSpec((1, HH), lambda i: (0, 0)),
            pl.BlockSpec((HH, 2), lambda i: (0, 0)),
            pl.BlockSpec((1, 2), lambda i: (0, 0)),
        ],
        out_specs=pl.BlockSpec((B_E, 2), lambda i: (i, 0)),
        out_shape=jax.ShapeDtypeStruct((E, 2), jnp.float32),
    )(xp, xp, b0.reshape(1, H), W1, b1.reshape(1, HH), W2, b2.reshape(1, 2))


def _edge_pass(Bh_t, Dh_t, Eh_t, ce, src, dst, want_epre):
    Bh = jnp.concatenate([Bh_t[:N], Bh_t[N:]], axis=1)
    Dh = jnp.concatenate([Dh_t[:N], Dh_t[N:]], axis=1)
    Eh = jnp.concatenate([Eh_t[:N], Eh_t[N:]], axis=1)
    ce_full = jnp.concatenate([ce[0], ce[1]], axis=1)
    ep_full = Dh[src] + Eh[dst] + ce_full
    sig = jax.nn.sigmoid(ep_full)
    num = jax.ops.segment_sum(Bh[src] * sig, dst, num_segments=N)
    den = jax.ops.segment_sum(sig, dst, num_segments=N)
    if want_epre:
        mu = jnp.mean(ep_full, axis=0).reshape(1, H)
        rstd = lax.rsqrt(jnp.var(ep_full, axis=0) + 1e-5).reshape(1, H)
        ep = jnp.stack([ep_full[:, :HH], ep_full[:, HH:]], axis=0)
    else:
        mu = rstd = ep = None
    return num, den, ep, mu, rstd


def _gather_pq(P_t, Q_t, src, dst):
    P = jnp.concatenate([P_t[:N], P_t[N:]], axis=1)
    Q = jnp.concatenate([Q_t[:N], Q_t[N:]], axis=1)
    xf = P[src] + Q[dst]
    return jnp.stack([xf[:, :HH], xf[:, HH:]], axis=0)


def kernel(h, e, edge_index, W_emb_h, b_emb_h, W_emb_e, b_emb_e, W_layers, b_layers,
           W_assign, b_assign, W_mlp0, b_mlp0, W_mlp1, b_mlp1, W_mlp2, b_mlp2):
    src = edge_index[0]
    dst = edge_index[1]

    hcur = _embed_h(h, W_emb_h, b_emb_h)
    ecur, ce = _edge_embed(e, W_emb_e, b_emb_e, W_layers[0, 2], b_layers[0, 2])

    s_list = []
    bi = 0
    for l in range(4):
        Wl, bl = W_layers[l], b_layers[l]
        Wstk = jnp.concatenate([Wl[0], Wl[1], Wl[3], Wl[4]], axis=1)
        bstk = jnp.concatenate([bl[0], bl[1], bl[3], bl[4]]).reshape(1, 4 * H)
        Ah, Bh_t, Dh_t, Eh_t = _node_mm(hcur, Wstk, bstk)
        num, den, ep, mu, rstd = _edge_pass(Bh_t, Dh_t, Eh_t, ce, src, dst,
                                            want_epre=(l < 3))
        hcur = _node_update(Ah, num, den, hcur)
        if l < 3:
            ecur, ce = _edge_update(ecur, ep, mu, rstd,
                                    W_layers[l + 1, 2], b_layers[l + 1, 2])
        if l in (1, 3):
            hcur, s = _bilinear(hcur, W_assign[bi], b_assign[bi])
            s_list.append(s)
            bi += 1

    S = jnp.stack(s_list, axis=0)
    P_t, Q_t = _readout_pq(hcur, W_mlp0)
    xp = _gather_pq(P_t, Q_t, src, dst)
    logits = _readout_mlp(xp, b_mlp0, W_mlp1, b_mlp1, W_mlp2, b_mlp2)
    return logits, S


# speedup vs baseline: 2.6976x; 2.6976x over previous
import functools

import jax
import jax.numpy as jnp
from jax import lax
from jax.experimental import pallas as pl
from jax.experimental.pallas import tpu as pltpu
from jax.experimental.pallas import tpu_sc as plsc

N = 10000
E = 320000
H = 128
A = 100
B_E = 8000

_NTILE = 16
_EPT = E // _NTILE
_K = 80
_NCHUNK = _EPT // _K
_NHALF = N // 2
_DEN_OFF = 5120
_TRASH = 10120
_ACC = 10240
_TROWS = _ACC // _NTILE
_EPT_PQ = E // 32


def _embed_h_body(x_ref, w_ref, b_ref, o_ref):
    o_ref[...] = jnp.dot(x_ref[...], w_ref[...], preferred_element_type=jnp.float32) + b_ref[...]


def _node_mm_body(x_ref, w_ref, b_ref, ah_ref, bh_ref, dh_ref, eh_ref):
    y = jnp.dot(x_ref[...], w_ref[...], preferred_element_type=jnp.float32) + b_ref[...]
    ah_ref[...] = y[:, 0:H]
    bh_ref[...] = y[:, H:2 * H]
    dh_ref[...] = y[:, 2 * H:3 * H]
    eh_ref[...] = y[:, 3 * H:4 * H]


def _node_update_body(ah_ref, num_ref, den_ref, hres_ref, o_ref):
    hn = ah_ref[...] + num_ref[...] / (den_ref[...] + 1e-6)
    mu = jnp.mean(hn, axis=0, keepdims=True)
    var = jnp.mean((hn - mu) ** 2, axis=0, keepdims=True)
    o_ref[...] = hres_ref[...] + jax.nn.relu((hn - mu) * lax.rsqrt(var + 1e-5))


def _edge_embed_body(e_ref, we_ref, be_ref, w2_ref, b2_ref, e0_ref, ce_ref):
    e0 = jnp.dot(e_ref[...], we_ref[...], preferred_element_type=jnp.float32) + be_ref[...]
    e0_ref[...] = e0
    ce_ref[...] = jnp.dot(e0, w2_ref[...], preferred_element_type=jnp.float32) + b2_ref[...]


def _edge_update_body(eres_ref, ep_ref, mu_ref, rstd_ref, w2_ref, b2_ref,
                      eo_ref, ce_ref):
    eo = eres_ref[...] + jax.nn.relu((ep_ref[...] - mu_ref[...]) * rstd_ref[...])
    eo_ref[...] = eo
    ce_ref[...] = jnp.dot(eo, w2_ref[...], preferred_element_type=jnp.float32) + b2_ref[...]


def _bilin_s_body(x_ref, wa_ref, ba_ref, s_ref):
    z = jnp.dot(x_ref[...], wa_ref[...], preferred_element_type=jnp.float32) + ba_ref[...]
    m = jnp.max(z, axis=-1, keepdims=True)
    ez = jnp.exp(z - m)
    s_ref[...] = ez / jnp.sum(ez, axis=-1, keepdims=True)


def _bilin_update_body(h_ref, s_ref, st_ref, o_ref):
    m = jnp.dot(st_ref[...], h_ref[...], preferred_element_type=jnp.float32)
    o_ref[...] = h_ref[...] + jnp.dot(s_ref[...], m, preferred_element_type=jnp.float32)


def _readout_pq_body(h_ref, w0_ref, p_ref, q_ref):
    p_ref[...] = jnp.dot(h_ref[...], w0_ref[0:H], preferred_element_type=jnp.float32)
    q_ref[...] = jnp.dot(h_ref[...], w0_ref[H:2 * H], preferred_element_type=jnp.float32)


def _mlp_body(xp_ref, b0_ref, w1_ref, b1_ref, w2_ref, b2_ref, o_ref):
    x = jax.nn.relu(xp_ref[...] + b0_ref[...])
    y = jax.nn.relu(jnp.dot(x, w1_ref[...], preferred_element_type=jnp.float32) + b1_ref[...])
    o_ref[...] = jnp.dot(y, w2_ref[...], preferred_element_type=jnp.float32) + b2_ref[...]


def _embed_h(h, W, b):
    return pl.pallas_call(
        _embed_h_body,
        out_shape=jax.ShapeDtypeStruct((N, H), jnp.float32),
    )(h, W, b.reshape(1, H))


def _node_mm(x, Wstk, bstk):
    return pl.pallas_call(
        _node_mm_body,
        out_shape=[jax.ShapeDtypeStruct((N, H), jnp.float32)] * 4,
    )(x, Wstk, bstk)


def _node_update(Ah, num, den, hres):
    return pl.pallas_call(
        _node_update_body,
        out_shape=jax.ShapeDtypeStruct((N, H), jnp.float32),
    )(Ah, num, den, hres)


def _edge_embed(e, Wemb, bemb, W2, b2):
    g = E // B_E
    return pl.pallas_call(
        _edge_embed_body,
        grid=(g,),
        in_specs=[
            pl.BlockSpec((B_E, 16), lambda i: (i, 0)),
            pl.BlockSpec((16, H), lambda i: (0, 0)),
            pl.BlockSpec((1, H), lambda i: (0, 0)),
            pl.BlockSpec((H, H), lambda i: (0, 0)),
            pl.BlockSpec((1, H), lambda i: (0, 0)),
        ],
        out_specs=[
            pl.BlockSpec((B_E, H), lambda i: (i, 0)),
            pl.BlockSpec((B_E, H), lambda i: (i, 0)),
        ],
        out_shape=[
            jax.ShapeDtypeStruct((E, H), jnp.float32),
            jax.ShapeDtypeStruct((E, H), jnp.float32),
        ],
    )(e, Wemb, bemb.reshape(1, H), W2, b2.reshape(1, H))


def _edge_update(eres, ep, mu, rstd, W2n, b2n):
    g = E // B_E
    return pl.pallas_call(
        _edge_update_body,
        grid=(g,),
        in_specs=[
            pl.BlockSpec((B_E, H), lambda i: (i, 0)),
            pl.BlockSpec((B_E, H), lambda i: (i, 0)),
            pl.BlockSpec((1, H), lambda i: (0, 0)),
            pl.BlockSpec((1, H), lambda i: (0, 0)),
            pl.BlockSpec((H, H), lambda i: (0, 0)),
            pl.BlockSpec((1, H), lambda i: (0, 0)),
        ],
        out_specs=[
            pl.BlockSpec((B_E, H), lambda i: (i, 0)),
            pl.BlockSpec((B_E, H), lambda i: (i, 0)),
        ],
        out_shape=[
            jax.ShapeDtypeStruct((E, H), jnp.float32),
            jax.ShapeDtypeStruct((E, H), jnp.float32),
        ],
    )(eres, ep, mu, rstd, W2n, b2n.reshape(1, H))


def _bilinear(h, Wa, ba):
    s = pl.pallas_call(
        _bilin_s_body,
        out_shape=jax.ShapeDtypeStruct((N, A), jnp.float32),
    )(h, Wa, ba.reshape(1, A))
    h_out = pl.pallas_call(
        _bilin_update_body,
        out_shape=jax.ShapeDtypeStruct((N, H), jnp.float32),
    )(h, s, s.T)
    return h_out, s


def _readout_pq(h, W0):
    return pl.pallas_call(
        _readout_pq_body,
        out_shape=[jax.ShapeDtypeStruct((N, H), jnp.float32)] * 2,
    )(h, W0)


def _readout_mlp(xp, b0, W1, b1, W2, b2):
    g = E // B_E
    return pl.pallas_call(
        _mlp_body,
        grid=(g,),
        in_specs=[
            pl.BlockSpec((B_E, H), lambda i: (i, 0)),
            pl.BlockSpec((1, H), lambda i: (0, 0)),
            pl.BlockSpec((H, H // 2), lambda i: (0, 0)),
            pl.BlockSpec((1, H // 2), lambda i: (0, 0)),
            pl.BlockSpec((H // 2, 2), lambda i: (0, 0)),
            pl.BlockSpec((1, 2), lambda i: (0, 0)),
        ],
        out_specs=pl.BlockSpec((B_E, 2), lambda i: (i, 0)),
        out_shape=jax.ShapeDtypeStruct((E, 2), jnp.float32),
    )(xp, b0.reshape(1, H), W1, b1.reshape(1, H // 2), W2, b2.reshape(1, 2))


def _sc_edge_body(want_epre, bh_hbm, dh_hbm, eh_hbm, ce_hbm, src_hbm, dst_hbm,
                  *refs):
    if want_epre:
        (nd_hbm, ep_hbm, st_hbm,
         srcv, dstv, snv, sdv,
         dhv, ehv, bhv, cev, statv, acc,
         s0, s1, s2, s3, s4, s5) = refs
    else:
        (nd_hbm,
         srcv, dstv, snv, sdv,
         dhv, ehv, bhv, cev, statv, acc,
         s0, s1, s2, s3, s4, s5) = refs
        ep_hbm = st_hbm = None
    cid = lax.axis_index("c")
    sid = lax.axis_index("s")

    def zrow(i, _):
        for j in range(8):
            dhv[i, pl.ds(j * 16, 16)] = jnp.zeros((16,), jnp.float32)
        return 0
    lax.fori_loop(0, _K, zrow, 0)
    for r in range(_TROWS // _K):
        pltpu.sync_copy(dhv, acc.at[pl.ds(sid * _TROWS + r * _K, _K)])
    plsc.subcore_barrier()

    base0 = sid * _EPT
    lo = cid * _NHALF

    def chunk(ic, carry):
        eb = base0 + ic * _K
        ca = pltpu.async_copy(src_hbm.at[pl.ds(eb, _K)], srcv, s0)
        cb = pltpu.async_copy(dst_hbm.at[pl.ds(eb, _K)], dstv, s1)
        ca.wait()
        cb.wait()
        for j in range(5):
            sl = pl.ds(j * 16, 16)
            dj = dstv[sl]
            mine = (dj >= lo) & (dj < lo + _NHALF)
            base = jnp.where(mine, dj - lo, _TRASH)
            snv[sl] = base
            sdv[sl] = jnp.where(mine, base + _DEN_OFF, _TRASH)
        c1 = pltpu.async_copy(dh_hbm.at[srcv], dhv, s2)
        c2 = pltpu.async_copy(eh_hbm.at[dstv], ehv, s3)
        c3 = pltpu.async_copy(bh_hbm.at[srcv], bhv, s4)
        c4 = pltpu.async_copy(ce_hbm.at[pl.ds(eb, _K)], cev, s5)
        c1.wait()
        c2.wait()
        c3.wait()
        c4.wait()

        def row(i, rc):
            out = list(rc)
            for j in range(8):
                sl = pl.ds(j * 16, 16)
                ep = dhv[i, sl] + ehv[i, sl] + cev[i, sl]
                sg = 1.0 / (1.0 + jnp.exp(-ep))
                bhv[i, sl] = bhv[i, sl] * sg
                ehv[i, sl] = sg
                if want_epre:
                    cev[i, sl] = ep
                    out[j] = rc[j] + ep
                    out[8 + j] = rc[8 + j] + ep * ep
            return tuple(out)
        carry = lax.fori_loop(0, _K, row, carry)

        w1 = pltpu.async_copy(bhv, acc.at[snv], s2, add=True)
        w2 = pltpu.async_copy(ehv, acc.at[sdv], s3, add=True)
        if want_epre:
            @pl.when((ic % 2) == cid)
            def _():
                pltpu.async_copy(cev, ep_hbm.at[pl.ds(eb, _K)], s4).wait()
        w1.wait()
        w2.wait()
        return carry

    zero16 = jnp.zeros((16,), jnp.float32)
    carry = lax.fori_loop(0, _NCHUNK, chunk, (zero16,) * 16)

    if want_epre:
        for j in range(8):
            sl = pl.ds(j * 16, 16)
            statv[0, sl] = carry[j]
            statv[1, sl] = carry[8 + j]
        pltpu.sync_copy(statv, st_hbm.at[pl.ds((cid * _NTILE + sid) * 8, 8)])

    plsc.subcore_barrier()
    pltpu.sync_copy(acc.at[pl.ds(sid * _TROWS, _TROWS)],
                    nd_hbm.at[pl.ds(cid * _ACC + sid * _TROWS, _TROWS)])


def _sc_edge(Bh, Dh, Eh, ce, src, dst, want_epre):
    mesh = plsc.VectorSubcoreMesh(core_axis_name="c", subcore_axis_name="s")
    out_type = [jax.ShapeDtypeStruct((2 * _ACC, H), jnp.float32)]
    if want_epre:
        out_type += [jax.ShapeDtypeStruct((E, H), jnp.float32),
                     jax.ShapeDtypeStruct((2 * _NTILE * 8, H), jnp.float32)]
    scratch = [
        pltpu.VMEM((_K,), jnp.int32),
        pltpu.VMEM((_K,), jnp.int32),
        pltpu.VMEM((_K,), jnp.int32),
        pltpu.VMEM((_K,), jnp.int32),
        pltpu.VMEM((_K, H), jnp.float32),
        pltpu.VMEM((_K, H), jnp.float32),
        pltpu.VMEM((_K, H), jnp.float32),
        pltpu.VMEM((_K, H), jnp.float32),
        pltpu.VMEM((8, H), jnp.float32),
        pltpu.VMEM_SHARED((_ACC, H), jnp.float32),
        pltpu.SemaphoreType.DMA,
        pltpu.SemaphoreType.DMA,
        pltpu.SemaphoreType.DMA,
        pltpu.SemaphoreType.DMA,
        pltpu.SemaphoreType.DMA,
        pltpu.SemaphoreType.DMA,
    ]
    outs = pl.kernel(
        functools.partial(_sc_edge_body, want_epre),
        out_type=out_type, mesh=mesh, scratch_types=scratch,
    )(Bh, Dh, Eh, ce, src, dst)
    if want_epre:
        nd, ep, st = outs
        st = st.reshape(2, _NTILE, 8, H)
        sums = st[:, :, 0].sum((0, 1))
        sqs = st[:, :, 1].sum((0, 1))
        mu_v = sums / (2 * E)
        mu = mu_v.reshape(1, H)
        rstd = lax.rsqrt(jnp.maximum(sqs / (2 * E) - mu_v ** 2, 0.0) + 1e-5).reshape(1, H)
    else:
        (nd,) = outs
        ep = mu = rstd = None
    nd = nd.reshape(2, _ACC, H)
    num = jnp.concatenate([nd[0, :_NHALF], nd[1, :_NHALF]], axis=0)
    den = jnp.concatenate([nd[0, _DEN_OFF:_DEN_OFF + _NHALF],
                           nd[1, _DEN_OFF:_DEN_OFF + _NHALF]], axis=0)
    return num, den, ep, mu, rstd


def _sc_pq_body(p_hbm, q_hbm, src_hbm, dst_hbm, xp_hbm,
                srcv, dstv, pv, qv, xv, s0, s1, s2, s3):
    cid = lax.axis_index("c")
    sid = lax.axis_index("s")
    base0 = (cid * _NTILE + sid) * _EPT_PQ

    def chunk(ic, _):
        eb = base0 + ic * _K
        ca = pltpu.async_copy(src_hbm.at[pl.ds(eb, _K)], srcv, s0)
        cb = pltpu.async_copy(dst_hbm.at[pl.ds(eb, _K)], dstv, s1)
        ca.wait()
        cb.wait()
        c1 = pltpu.async_copy(p_hbm.at[srcv], pv, s2)
        c2 = pltpu.async_copy(q_hbm.at[dstv], qv, s3)
        c1.wait()
        c2.wait()

        def row(i, _):
            for j in range(8):
                sl = pl.ds(j * 16, 16)
                xv[i, sl] = pv[i, sl] + qv[i, sl]
            return 0
        lax.fori_loop(0, _K, row, 0)
        pltpu.sync_copy(xv, xp_hbm.at[pl.ds(eb, _K)])
        return 0

    lax.fori_loop(0, _EPT_PQ // _K, chunk, 0)


def _sc_pq(P, Q, src, dst):
    mesh = plsc.VectorSubcoreMesh(core_axis_name="c", subcore_axis_name="s")
    scratch = [
        pltpu.VMEM((_K,), jnp.int32),
        pltpu.VMEM((_K,), jnp.int32),
        pltpu.VMEM((_K, H), jnp.float32),
        pltpu.VMEM((_K, H), jnp.float32),
        pltpu.VMEM((_K, H), jnp.float32),
        pltpu.SemaphoreType.DMA,
        pltpu.SemaphoreType.DMA,
        pltpu.SemaphoreType.DMA,
        pltpu.SemaphoreType.DMA,
    ]
    return pl.kernel(
        _sc_pq_body,
        out_type=jax.ShapeDtypeStruct((E, H), jnp.float32),
        mesh=mesh, scratch_types=scratch,
    )(P, Q, src, dst)


def kernel(h, e, edge_index, W_emb_h, b_emb_h, W_emb_e, b_emb_e, W_layers, b_layers,
           W_assign, b_assign, W_mlp0, b_mlp0, W_mlp1, b_mlp1, W_mlp2, b_mlp2):
    src = edge_index[0]
    dst = edge_index[1]

    hcur = _embed_h(h, W_emb_h, b_emb_h)
    ecur, ce = _edge_embed(e, W_emb_e, b_emb_e, W_layers[0, 2], b_layers[0, 2])

    s_list = []
    bi = 0
    for l in range(4):
        Wl, bl = W_layers[l], b_layers[l]
        Wstk = jnp.concatenate([Wl[0], Wl[1], Wl[3], Wl[4]], axis=1)
        bstk = jnp.concatenate([bl[0], bl[1], bl[3], bl[4]]).reshape(1, 4 * H)
        Ah, Bh, Dh, Eh = _node_mm(hcur, Wstk, bstk)
        num, den, ep, mu, rstd = _sc_edge(Bh, Dh, Eh, ce, src, dst,
                                          want_epre=(l < 3))
        hcur = _node_update(Ah, num, den, hcur)
        if l < 3:
            ecur, ce = _edge_update(ecur, ep, mu, rstd,
                                    W_layers[l + 1, 2], b_layers[l + 1, 2])
        if l in (1, 3):
            hcur, s = _bilinear(hcur, W_assign[bi], b_assign[bi])
            s_list.append(s)
            bi += 1

    S = jnp.stack(s_list, axis=0)
    P, Q = _readout_pq(hcur, W_mlp0)
    xp = _sc_pq(P, Q, src, dst)
    logits = _readout_mlp(xp, b_mlp0, W_mlp1, b_mlp1, W_mlp2, b_mlp2)
    return logits, S


# speedup vs baseline: 3.7641x; 1.3953x over previous
import functools

import jax
import jax.numpy as jnp
from jax import lax
from jax.experimental import pallas as pl
from jax.experimental.pallas import tpu as pltpu
from jax.experimental.pallas import tpu_sc as plsc

N = 10000
E = 320000
H = 128
A = 100
B_E = 8000

_NTILE = 16
_EPT = E // _NTILE
_K = 40
_NCHUNK = _EPT // _K
_NB2 = _NCHUNK // 2
_KPQ = 80
_NHALF = N // 2
_DEN_OFF = 5120
_TRASH = 10120
_ACC = 10240
_TROWS = _ACC // _NTILE
_EPT_PQ = E // 32


def _embed_h_body(x_ref, w_ref, b_ref, o_ref):
    o_ref[...] = jnp.dot(x_ref[...], w_ref[...], preferred_element_type=jnp.float32) + b_ref[...]


def _node_mm_body(x_ref, w_ref, b_ref, ah_ref, bh_ref, dh_ref, eh_ref):
    y = jnp.dot(x_ref[...], w_ref[...], preferred_element_type=jnp.float32) + b_ref[...]
    ah_ref[...] = y[:, 0:H]
    bh_ref[...] = y[:, H:2 * H]
    dh_ref[...] = y[:, 2 * H:3 * H]
    eh_ref[...] = y[:, 3 * H:4 * H]


def _node_update_body(ah_ref, num_ref, den_ref, hres_ref, o_ref):
    hn = ah_ref[...] + num_ref[...] / (den_ref[...] + 1e-6)
    mu = jnp.mean(hn, axis=0, keepdims=True)
    var = jnp.mean((hn - mu) ** 2, axis=0, keepdims=True)
    o_ref[...] = hres_ref[...] + jax.nn.relu((hn - mu) * lax.rsqrt(var + 1e-5))


def _edge_embed_body(e_ref, we_ref, be_ref, w2_ref, b2_ref, e0_ref, ce_ref):
    e0 = jnp.dot(e_ref[...], we_ref[...], preferred_element_type=jnp.float32) + be_ref[...]
    e0_ref[...] = e0
    ce_ref[...] = jnp.dot(e0, w2_ref[...], preferred_element_type=jnp.float32) + b2_ref[...]


def _edge_update_body(eres_ref, ep_ref, mu_ref, rstd_ref, w2_ref, b2_ref,
                      eo_ref, ce_ref):
    eo = eres_ref[...] + jax.nn.relu((ep_ref[...] - mu_ref[...]) * rstd_ref[...])
    eo_ref[...] = eo
    ce_ref[...] = jnp.dot(eo, w2_ref[...], preferred_element_type=jnp.float32) + b2_ref[...]


def _bilin_s_body(x_ref, wa_ref, ba_ref, s_ref):
    z = jnp.dot(x_ref[...], wa_ref[...], preferred_element_type=jnp.float32) + ba_ref[...]
    m = jnp.max(z, axis=-1, keepdims=True)
    ez = jnp.exp(z - m)
    s_ref[...] = ez / jnp.sum(ez, axis=-1, keepdims=True)


def _bilin_update_body(h_ref, s_ref, st_ref, o_ref):
    m = jnp.dot(st_ref[...], h_ref[...], preferred_element_type=jnp.float32)
    o_ref[...] = h_ref[...] + jnp.dot(s_ref[...], m, preferred_element_type=jnp.float32)


def _readout_pq_body(h_ref, w0_ref, p_ref, q_ref):
    p_ref[...] = jnp.dot(h_ref[...], w0_ref[0:H], preferred_element_type=jnp.float32)
    q_ref[...] = jnp.dot(h_ref[...], w0_ref[H:2 * H], preferred_element_type=jnp.float32)


def _mlp_body(xp_ref, b0_ref, w1_ref, b1_ref, w2_ref, b2_ref, o_ref):
    x = jax.nn.relu(xp_ref[...] + b0_ref[...])
    y = jax.nn.relu(jnp.dot(x, w1_ref[...], preferred_element_type=jnp.float32) + b1_ref[...])
    o_ref[...] = jnp.dot(y, w2_ref[...], preferred_element_type=jnp.float32) + b2_ref[...]


def _embed_h(h, W, b):
    return pl.pallas_call(
        _embed_h_body,
        out_shape=jax.ShapeDtypeStruct((N, H), jnp.float32),
    )(h, W, b.reshape(1, H))


def _node_mm(x, Wstk, bstk):
    return pl.pallas_call(
        _node_mm_body,
        out_shape=[jax.ShapeDtypeStruct((N, H), jnp.float32)] * 4,
    )(x, Wstk, bstk)


def _node_update(Ah, num, den, hres):
    return pl.pallas_call(
        _node_update_body,
        out_shape=jax.ShapeDtypeStruct((N, H), jnp.float32),
    )(Ah, num, den, hres)


def _edge_embed(e, Wemb, bemb, W2, b2):
    g = E // B_E
    return pl.pallas_call(
        _edge_embed_body,
        grid=(g,),
        in_specs=[
            pl.BlockSpec((B_E, 16), lambda i: (i, 0)),
            pl.BlockSpec((16, H), lambda i: (0, 0)),
            pl.BlockSpec((1, H), lambda i: (0, 0)),
            pl.BlockSpec((H, H), lambda i: (0, 0)),
            pl.BlockSpec((1, H), lambda i: (0, 0)),
        ],
        out_specs=[
            pl.BlockSpec((B_E, H), lambda i: (i, 0)),
            pl.BlockSpec((B_E, H), lambda i: (i, 0)),
        ],
        out_shape=[
            jax.ShapeDtypeStruct((E, H), jnp.float32),
            jax.ShapeDtypeStruct((E, H), jnp.float32),
        ],
    )(e, Wemb, bemb.reshape(1, H), W2, b2.reshape(1, H))


def _edge_update(eres, ep, mu, rstd, W2n, b2n):
    g = E // B_E
    return pl.pallas_call(
        _edge_update_body,
        grid=(g,),
        in_specs=[
            pl.BlockSpec((B_E, H), lambda i: (i, 0)),
            pl.BlockSpec((B_E, H), lambda i: (i, 0)),
            pl.BlockSpec((1, H), lambda i: (0, 0)),
            pl.BlockSpec((1, H), lambda i: (0, 0)),
            pl.BlockSpec((H, H), lambda i: (0, 0)),
            pl.BlockSpec((1, H), lambda i: (0, 0)),
        ],
        out_specs=[
            pl.BlockSpec((B_E, H), lambda i: (i, 0)),
            pl.BlockSpec((B_E, H), lambda i: (i, 0)),
        ],
        out_shape=[
            jax.ShapeDtypeStruct((E, H), jnp.float32),
            jax.ShapeDtypeStruct((E, H), jnp.float32),
        ],
    )(eres, ep, mu, rstd, W2n, b2n.reshape(1, H))


def _bilinear(h, Wa, ba):
    s = pl.pallas_call(
        _bilin_s_body,
        out_shape=jax.ShapeDtypeStruct((N, A), jnp.float32),
    )(h, Wa, ba.reshape(1, A))
    h_out = pl.pallas_call(
        _bilin_update_body,
        out_shape=jax.ShapeDtypeStruct((N, H), jnp.float32),
    )(h, s, s.T)
    return h_out, s


def _readout_pq(h, W0):
    return pl.pallas_call(
        _readout_pq_body,
        out_shape=[jax.ShapeDtypeStruct((N, H), jnp.float32)] * 2,
    )(h, W0)


def _readout_mlp(xp, b0, W1, b1, W2, b2):
    g = E // B_E
    return pl.pallas_call(
        _mlp_body,
        grid=(g,),
        in_specs=[
            pl.BlockSpec((B_E, H), lambda i: (i, 0)),
            pl.BlockSpec((1, H), lambda i: (0, 0)),
            pl.BlockSpec((H, H // 2), lambda i: (0, 0)),
            pl.BlockSpec((1, H // 2), lambda i: (0, 0)),
            pl.BlockSpec((H // 2, 2), lambda i: (0, 0)),
            pl.BlockSpec((1, 2), lambda i: (0, 0)),
        ],
        out_specs=pl.BlockSpec((B_E, 2), lambda i: (i, 0)),
        out_shape=jax.ShapeDtypeStruct((E, 2), jnp.float32),
    )(xp, b0.reshape(1, H), W1, b1.reshape(1, H // 2), W2, b2.reshape(1, 2))


def _sc_edge_body(want_epre, bh_hbm, dh_hbm, eh_hbm, ce_hbm, src_hbm, dst_hbm,
                  *refs):
    if want_epre:
        nd_hbm, ep_hbm, st_hbm = refs[:3]
        r = refs[3:]
    else:
        nd_hbm = refs[0]
        ep_hbm = st_hbm = None
        r = refs[1:]
    slots = (r[0:8], r[8:16])
    statv, acc = r[16], r[17]
    si = (r[18], r[19])
    sg = (r[20], r[21])
    sw = (r[22], r[23])
    sep = (r[24], r[25])
    cid = lax.axis_index("c")
    sid = lax.axis_index("s")

    dhv0 = slots[0][4]

    def zrow(i, _):
        for j in range(8):
            dhv0[i, pl.ds(j * 16, 16)] = jnp.zeros((16,), jnp.float32)
        return 0
    lax.fori_loop(0, _K, zrow, 0)
    for rr in range(_TROWS // _K):
        pltpu.sync_copy(dhv0, acc.at[pl.ds(sid * _TROWS + rr * _K, _K)])
    plsc.subcore_barrier()

    base0 = sid * _EPT
    lo = cid * _NHALF

    def fire_idx(eb, s):
        srcv, dstv = slots[s][0], slots[s][1]
        pltpu.async_copy(src_hbm.at[pl.ds(eb, _K)], srcv, si[s])
        pltpu.async_copy(dst_hbm.at[pl.ds(eb, _K)], dstv, si[s])

    def wait_idx(s):
        srcv, dstv = slots[s][0], slots[s][1]
        pltpu.make_async_copy(src_hbm.at[pl.ds(0, _K)], srcv, si[s]).wait()
        pltpu.make_async_copy(src_hbm.at[pl.ds(0, _K)], dstv, si[s]).wait()

    def prep(s):
        srcv, dstv, snv, sdv = slots[s][0:4]
        for off in (0, 16, 24):
            sl = pl.ds(off, 16)
            dj = dstv[sl]
            mine = (dj >= lo) & (dj < lo + _NHALF)
            base = jnp.where(mine, dj - lo, _TRASH)
            snv[sl] = base
            sdv[sl] = jnp.where(mine, base + _DEN_OFF, _TRASH)

    def fire_gather(eb, s):
        srcv, dstv, _, _, dhv, ehv, bhv, cev = slots[s]
        pltpu.async_copy(dh_hbm.at[srcv], dhv, sg[s])
        pltpu.async_copy(eh_hbm.at[dstv], ehv, sg[s])
        pltpu.async_copy(bh_hbm.at[srcv], bhv, sg[s])
        pltpu.async_copy(ce_hbm.at[pl.ds(eb, _K)], cev, sg[s])

    def wait_gather(s):
        dhv = slots[s][4]
        for _ in range(4):
            pltpu.make_async_copy(ce_hbm.at[pl.ds(0, _K)], dhv, sg[s]).wait()

    def compute(s, carry):
        _, _, _, _, dhv, ehv, bhv, cev = slots[s]

        def row(i, rc):
            out = list(rc)
            for j in range(8):
                sl = pl.ds(j * 16, 16)
                ep = dhv[i, sl] + ehv[i, sl] + cev[i, sl]
                sgm = 1.0 / (1.0 + jnp.exp(-ep))
                bhv[i, sl] = bhv[i, sl] * sgm
                ehv[i, sl] = sgm
                if want_epre:
                    cev[i, sl] = ep
                    out[j] = rc[j] + ep
                    out[8 + j] = rc[8 + j] + ep * ep
            return tuple(out)
        return lax.fori_loop(0, _K, row, carry)

    def fire_scatter(eb, s):
        _, _, snv, sdv, dhv, ehv, bhv, cev = slots[s]
        pltpu.async_copy(bhv, acc.at[snv], sw[s], add=True)
        pltpu.async_copy(ehv, acc.at[sdv], sw[s], add=True)
        if want_epre:
            @pl.when(cid == s)
            def _():
                pltpu.async_copy(cev, ep_hbm.at[pl.ds(eb, _K)], sep[s])

    def wait_scatter(s):
        dhv = slots[s][4]
        for _ in range(2):
            pltpu.make_async_copy(ce_hbm.at[pl.ds(0, _K)], dhv, sw[s]).wait()
        if want_epre:
            @pl.when(cid == s)
            def _():
                pltpu.make_async_copy(ce_hbm.at[pl.ds(0, _K)], dhv, sep[s]).wait()

    fire_idx(base0, 0)
    fire_idx(base0 + _K, 1)
    wait_idx(0)
    prep(0)
    fire_gather(base0, 0)
    wait_idx(1)
    prep(1)
    fire_gather(base0 + _K, 1)

    def body(t, carry):
        for s in (0, 1):
            eb = base0 + (2 * t + s) * _K
            wait_gather(s)

            @pl.when(t < _NB2 - 1)
            def _():
                fire_idx(eb + 2 * _K, s)
            carry = compute(s, carry)
            fire_scatter(eb, s)
            wait_scatter(s)

            @pl.when(t < _NB2 - 1)
            def _():
                wait_idx(s)
                prep(s)
                fire_gather(eb + 2 * _K, s)
        return carry

    zero16 = jnp.zeros((16,), jnp.float32)
    carry = lax.fori_loop(0, _NB2, body, (zero16,) * 16)

    if want_epre:
        for j in range(8):
            sl = pl.ds(j * 16, 16)
            statv[0, sl] = carry[j]
            statv[1, sl] = carry[8 + j]
        pltpu.sync_copy(statv, st_hbm.at[pl.ds((cid * _NTILE + sid) * 8, 8)])

    plsc.subcore_barrier()
    pltpu.sync_copy(acc.at[pl.ds(sid * _TROWS, _TROWS)],
                    nd_hbm.at[pl.ds(cid * _ACC + sid * _TROWS, _TROWS)])


def _sc_edge(Bh, Dh, Eh, ce, src, dst, want_epre):
    mesh = plsc.VectorSubcoreMesh(core_axis_name="c", subcore_axis_name="s")
    out_type = [jax.ShapeDtypeStruct((2 * _ACC, H), jnp.float32)]
    if want_epre:
        out_type += [jax.ShapeDtypeStruct((E, H), jnp.float32),
                     jax.ShapeDtypeStruct((2 * _NTILE * 8, H), jnp.float32)]
    scratch = []
    for _s in range(2):
        scratch += [
            pltpu.VMEM((_K,), jnp.int32),
            pltpu.VMEM((_K,), jnp.int32),
            pltpu.VMEM((_K,), jnp.int32),
            pltpu.VMEM((_K,), jnp.int32),
            pltpu.VMEM((_K, H), jnp.float32),
            pltpu.VMEM((_K, H), jnp.float32),
            pltpu.VMEM((_K, H), jnp.float32),
            pltpu.VMEM((_K, H), jnp.float32),
        ]
    scratch += [
        pltpu.VMEM((8, H), jnp.float32),
        pltpu.VMEM_SHARED((_ACC, H), jnp.float32),
    ]
    scratch += [pltpu.SemaphoreType.DMA] * 8
    outs = pl.kernel(
        functools.partial(_sc_edge_body, want_epre),
        out_type=out_type, mesh=mesh, scratch_types=scratch,
    )(Bh, Dh, Eh, ce, src, dst)
    if want_epre:
        nd, ep, st = outs
        st = st.reshape(2, _NTILE, 8, H)
        sums = st[:, :, 0].sum((0, 1))
        sqs = st[:, :, 1].sum((0, 1))
        mu_v = sums / (2 * E)
        mu = mu_v.reshape(1, H)
        rstd = lax.rsqrt(jnp.maximum(sqs / (2 * E) - mu_v ** 2, 0.0) + 1e-5).reshape(1, H)
    else:
        (nd,) = outs
        ep = mu = rstd = None
    nd = nd.reshape(2, _ACC, H)
    num = jnp.concatenate([nd[0, :_NHALF], nd[1, :_NHALF]], axis=0)
    den = jnp.concatenate([nd[0, _DEN_OFF:_DEN_OFF + _NHALF],
                           nd[1, _DEN_OFF:_DEN_OFF + _NHALF]], axis=0)
    return num, den, ep, mu, rstd


def _sc_pq_body(p_hbm, q_hbm, src_hbm, dst_hbm, xp_hbm,
                srcv, dstv, pv, qv, xv, s0, s1, s2, s3):
    cid = lax.axis_index("c")
    sid = lax.axis_index("s")
    base0 = (cid * _NTILE + sid) * _EPT_PQ

    def chunk(ic, _):
        eb = base0 + ic * _KPQ
        ca = pltpu.async_copy(src_hbm.at[pl.ds(eb, _KPQ)], srcv, s0)
        cb = pltpu.async_copy(dst_hbm.at[pl.ds(eb, _KPQ)], dstv, s1)
        ca.wait()
        cb.wait()
        c1 = pltpu.async_copy(p_hbm.at[srcv], pv, s2)
        c2 = pltpu.async_copy(q_hbm.at[dstv], qv, s3)
        c1.wait()
        c2.wait()

        def row(i, _):
            for j in range(8):
                sl = pl.ds(j * 16, 16)
                xv[i, sl] = pv[i, sl] + qv[i, sl]
            return 0
        lax.fori_loop(0, _KPQ, row, 0)
        pltpu.sync_copy(xv, xp_hbm.at[pl.ds(eb, _KPQ)])
        return 0

    lax.fori_loop(0, _EPT_PQ // _KPQ, chunk, 0)


def _sc_pq(P, Q, src, dst):
    mesh = plsc.VectorSubcoreMesh(core_axis_name="c", subcore_axis_name="s")
    scratch = [
        pltpu.VMEM((_KPQ,), jnp.int32),
        pltpu.VMEM((_KPQ,), jnp.int32),
        pltpu.VMEM((_KPQ, H), jnp.float32),
        pltpu.VMEM((_KPQ, H), jnp.float32),
        pltpu.VMEM((_KPQ, H), jnp.float32),
        pltpu.SemaphoreType.DMA,
        pltpu.SemaphoreType.DMA,
        pltpu.SemaphoreType.DMA,
        pltpu.SemaphoreType.DMA,
    ]
    return pl.kernel(
        _sc_pq_body,
        out_type=jax.ShapeDtypeStruct((E, H), jnp.float32),
        mesh=mesh, scratch_types=scratch,
    )(P, Q, src, dst)


def kernel(h, e, edge_index, W_emb_h, b_emb_h, W_emb_e, b_emb_e, W_layers, b_layers,
           W_assign, b_assign, W_mlp0, b_mlp0, W_mlp1, b_mlp1, W_mlp2, b_mlp2):
    src = edge_index[0]
    dst = edge_index[1]

    hcur = _embed_h(h, W_emb_h, b_emb_h)
    ecur, ce = _edge_embed(e, W_emb_e, b_emb_e, W_layers[0, 2], b_layers[0, 2])

    s_list = []
    bi = 0
    for l in range(4):
        Wl, bl = W_layers[l], b_layers[l]
        Wstk = jnp.concatenate([Wl[0], Wl[1], Wl[3], Wl[4]], axis=1)
        bstk = jnp.concatenate([bl[0], bl[1], bl[3], bl[4]]).reshape(1, 4 * H)
        Ah, Bh, Dh, Eh = _node_mm(hcur, Wstk, bstk)
        num, den, ep, mu, rstd = _sc_edge(Bh, Dh, Eh, ce, src, dst,
                                          want_epre=(l < 3))
        hcur = _node_update(Ah, num, den, hcur)
        if l < 3:
            ecur, ce = _edge_update(ecur, ep, mu, rstd,
                                    W_layers[l + 1, 2], b_layers[l + 1, 2])
        if l in (1, 3):
            hcur, s = _bilinear(hcur, W_assign[bi], b_assign[bi])
            s_list.append(s)
            bi += 1

    S = jnp.stack(s_list, axis=0)
    P, Q = _readout_pq(hcur, W_mlp0)
    xp = _sc_pq(P, Q, src, dst)
    logits = _readout_mlp(xp, b_mlp0, W_mlp1, b_mlp1, W_mlp2, b_mlp2)
    return logits, S
